# baseline (speedup 1.0000x reference)
import jax
import jax.numpy as jnp
from jax import lax
from jax.experimental import pallas as pl
from jax.experimental.pallas import tpu as pltpu
from jax.experimental.pallas import tpu_sc as plsc

N = 10000
E = 320000
H = 128
NUM_RBF = 32
N_PAD = 10240

NC, NS = 2, 16
NW = NC * NS
CH = 128
POS_W = 4

EPT1 = 4992
EPT2 = 5008
E1 = EPT1 * NW
E2 = EPT2 * NW
BLK1 = 2048
BLK2 = 2504


def _silu(x):
    return x * (1.0 / (1.0 + jnp.exp(-x)))


def _stage_a_body(h_ref, w1a_ref, w1b_ref, emb_ref, w1c_ref, b1_ref,
                  ha_ref, hb_ref, embc_ref):
    hv = h_ref[...]
    ha_ref[...] = jnp.dot(hv, w1a_ref[...], preferred_element_type=jnp.float32)
    hb_ref[...] = jnp.dot(hv, w1b_ref[...], preferred_element_type=jnp.float32)
    embc_ref[...] = (
        jnp.dot(emb_ref[...], w1c_ref[...], preferred_element_type=jnp.float32)
        + b1_ref[...])


def _stage_a(h, w1a, w1b, emb, w1c, b1):
    blk = 2000
    grid = N // blk
    return pl.pallas_call(
        _stage_a_body,
        grid=(grid,),
        in_specs=[
            pl.BlockSpec((blk, H), lambda i: (i, 0)),
            pl.BlockSpec((H, H), lambda i: (0, 0)),
            pl.BlockSpec((H, H), lambda i: (0, 0)),
            pl.BlockSpec((2, H), lambda i: (0, 0)),
            pl.BlockSpec((H, H), lambda i: (0, 0)),
            pl.BlockSpec((1, H), lambda i: (0, 0)),
        ],
        out_specs=[
            pl.BlockSpec((blk, H), lambda i: (i, 0)),
            pl.BlockSpec((blk, H), lambda i: (i, 0)),
            pl.BlockSpec((2, H), lambda i: (0, 0)),
        ],
        out_shape=[
            jax.ShapeDtypeStruct((N, H), jnp.float32),
            jax.ShapeDtypeStruct((N, H), jnp.float32),
            jax.ShapeDtypeStruct((2, H), jnp.float32),
        ],
    )(h, w1a, w1b, emb, w1c, b1)


def _make_stage_b(ept, e0):
    nfull = ept // CH
    nf = nfull - (nfull % 3)
    extra = nfull - nf
    tail = ept - nfull * CH
    ne = ept * NW

    def body(ha_hbm, hb_hbm, posflat_hbm, src_hbm, dst_hbm,
             preab_hbm, d2_hbm,
             idx_s0, idx_s1, idx_s2, idx_d0, idx_d1, idx_d2,
             idx_s_t, idx_d_t,
             rows0, rows1, rows2, d2b0, d2b1, d2b2, posv,
             sga0, sga1, sga2, sgb0, sgb1, sgb2, sst0, sst1, sst2):
        wid = lax.axis_index("c") * NS + lax.axis_index("s")
        lbase0 = wid * ept
        gbase0 = e0 + lbase0
        idx_s = (idx_s0, idx_s1, idx_s2)
        idx_d = (idx_d0, idx_d1, idx_d2)
        rows = (rows0, rows1, rows2)
        d2b = (d2b0, d2b1, d2b2)
        sga = (sga0, sga1, sga2)
        sgb = (sgb0, sgb1, sgb2)
        sst = (sst0, sst1, sst2)

        pltpu.sync_copy(posflat_hbm, posv)

        def dist2_chunk(si, di, out, n_groups):
            for g in range(n_groups):
                s16 = si[pl.ds(g * 16, 16)]
                d16 = di[pl.ds(g * 16, 16)]
                sb = s16 * POS_W
                db = d16 * POS_W
                acc = jnp.zeros((16,), jnp.float32)
                for k in range(3):
                    a = plsc.load_gather(posv, [sb + k])
                    bb = plsc.load_gather(posv, [db + k])
                    r = a - bb
                    acc = acc + r * r
                out[pl.ds(g * 16, 16)] = acc

        def issue_store(j, slot):
            base = lbase0 + j * CH
            pltpu.async_copy(rows[slot], preab_hbm.at[pl.ds(base, CH)],
                             sst[slot])
            pltpu.async_copy(d2b[slot], d2_hbm.at[pl.ds(base, CH)], sst[slot])

        def wait_store(j, slot):
            base = lbase0 + j * CH
            pltpu.make_async_copy(rows[slot], preab_hbm.at[pl.ds(base, CH)],
                                  sst[slot]).wait()
            pltpu.make_async_copy(d2b[slot], d2_hbm.at[pl.ds(base, CH)],
                                  sst[slot]).wait()

        def wait_ga(slot):
            pltpu.make_async_copy(ha_hbm.at[idx_s[slot]], rows[slot],
                                  sga[slot]).wait()

        def wait_gb(slot):
            pltpu.make_async_copy(hb_hbm.at[idx_d[slot]], rows[slot],
                                  sgb[slot]).wait()

        def load_idx_and_ga(j, slot):
            gb = gbase0 + j * CH
            pltpu.sync_copy(src_hbm.at[pl.ds(gb, CH)], idx_s[slot])
            pltpu.sync_copy(dst_hbm.at[pl.ds(gb, CH)], idx_d[slot])
            pltpu.async_copy(ha_hbm.at[idx_s[slot]], rows[slot], sga[slot])

        load_idx_and_ga(0, 0)

        def triple(g, carry):
            for b in (0, 1, 2):
                i = g * 3 + b
                p1 = (b - 1) % 3
                p2 = (b - 2) % 3
                wait_ga(b)
                pltpu.async_copy(hb_hbm.at[idx_d[b]], rows[b], sgb[b],
                                 add=True)
                dist2_chunk(idx_s[b], idx_d[b], d2b[b], CH // 16)
                if b == 0:
                    @pl.when(g > 0)
                    def _():
                        wait_gb(p1)
                        issue_store(i - 1, p1)
                        wait_store(i - 2, p2)
                    load_idx_and_ga(i + 1, p2)
                elif b == 1:
                    wait_gb(p1)
                    issue_store(i - 1, p1)

                    @pl.when(g > 0)
                    def _():
                        wait_store(i - 2, p2)
                    load_idx_and_ga(i + 1, p2)
                else:
                    wait_gb(p1)
                    issue_store(i - 1, p1)
                    wait_store(i - 2, p2)

                    @pl.when(g < nf // 3 - 1)
                    def _():
                        load_idx_and_ga(i + 1, p2)
            return carry

        lax.fori_loop(0, nf // 3, triple, 0)

        last = nf - 1
        wait_gb(last % 3)
        issue_store(last, last % 3)
        wait_store(last - 1, (last - 1) % 3)
        wait_store(last, last % 3)

        for x in range(extra):
            gb = gbase0 + (nf + x) * CH
            lb = lbase0 + (nf + x) * CH
            pltpu.sync_copy(src_hbm.at[pl.ds(gb, CH)], idx_s[0])
            pltpu.sync_copy(dst_hbm.at[pl.ds(gb, CH)], idx_d[0])
            pltpu.async_copy(ha_hbm.at[idx_s[0]], rows[0], sga[0]).wait()
            cp = pltpu.async_copy(hb_hbm.at[idx_d[0]], rows[0], sga[0],
                                  add=True)
            dist2_chunk(idx_s[0], idx_d[0], d2b[0], CH // 16)
            cp.wait()
            pltpu.sync_copy(rows[0], preab_hbm.at[pl.ds(lb, CH)])
            pltpu.sync_copy(d2b[0], d2_hbm.at[pl.ds(lb, CH)])

        if tail:
            gt = gbase0 + nfull * CH
            lt = lbase0 + nfull * CH
            pltpu.sync_copy(src_hbm.at[pl.ds(gt, tail)], idx_s_t)
            pltpu.sync_copy(dst_hbm.at[pl.ds(gt, tail)], idx_d_t)
            pltpu.async_copy(ha_hbm.at[idx_s_t], rows0.at[pl.ds(0, tail)],
                             sga0).wait()
            cp = pltpu.async_copy(hb_hbm.at[idx_d_t], rows0.at[pl.ds(0, tail)],
                                  sga0, add=True)
            dist2_chunk(idx_s_t, idx_d_t, d2b0, tail // 16)
            cp.wait()
            pltpu.sync_copy(rows0.at[pl.ds(0, tail)],
                            preab_hbm.at[pl.ds(lt, tail)])
            pltpu.sync_copy(d2b0.at[pl.ds(0, tail)], d2_hbm.at[pl.ds(lt, tail)])

    mesh = plsc.VectorSubcoreMesh(core_axis_name="c", subcore_axis_name="s",
                                  num_cores=NC, num_subcores=NS)
    tshape = max(tail, 16)
    return pl.kernel(
        body,
        out_type=[
            jax.ShapeDtypeStruct((ne, H), jnp.float32),
            jax.ShapeDtypeStruct((ne,), jnp.float32),
        ],
        mesh=mesh,
        scratch_types=[
            pltpu.VMEM((CH,), jnp.int32),
            pltpu.VMEM((CH,), jnp.int32),
            pltpu.VMEM((CH,), jnp.int32),
            pltpu.VMEM((CH,), jnp.int32),
            pltpu.VMEM((CH,), jnp.int32),
            pltpu.VMEM((CH,), jnp.int32),
            pltpu.VMEM((tshape,), jnp.int32),
            pltpu.VMEM((tshape,), jnp.int32),
            pltpu.VMEM((CH, H), jnp.float32),
            pltpu.VMEM((CH, H), jnp.float32),
            pltpu.VMEM((CH, H), jnp.float32),
            pltpu.VMEM((CH,), jnp.float32),
            pltpu.VMEM((CH,), jnp.float32),
            pltpu.VMEM((CH,), jnp.float32),
            pltpu.VMEM((N * POS_W,), jnp.float32),
            pltpu.SemaphoreType.DMA,
            pltpu.SemaphoreType.DMA,
            pltpu.SemaphoreType.DMA,
            pltpu.SemaphoreType.DMA,
            pltpu.SemaphoreType.DMA,
            pltpu.SemaphoreType.DMA,
            pltpu.SemaphoreType.DMA,
            pltpu.SemaphoreType.DMA,
            pltpu.SemaphoreType.DMA,
        ],
        compiler_params=pltpu.CompilerParams(needs_layout_passes=False),
    )


C_BLK = 6400
C_G = C_BLK // 128


def _stage_c_body(preab_ref, d2g_ref, etg_ref, dstg_ref, wext_ref,
                  censg_ref, gam_ref, w2_ref, b2_ref, m_ref, cnt_ref):
    i = pl.program_id(0)
    w2 = w2_ref[...]
    b2 = b2_ref[...]
    wext = wext_ref[...]
    censg = censg_ref[...]
    gam = gam_ref[...]
    liota = lax.broadcasted_iota(jnp.int32, (128, 128), 1).astype(jnp.float32)
    ones_col = jnp.ones((128, 1), jnp.float32)

    d2t = d2g_ref[0].T
    ett = etg_ref[0].T
    dstt = dstg_ref[0].T

    acc = jnp.zeros((128, 128), jnp.float32)
    for g in range(C_G):
        d2c = d2t[:, g:g + 1]
        dist = jnp.sqrt(d2c * gam)
        diff = dist - censg
        radial = jnp.exp(-diff * diff)
        etc = ett[:, g:g + 1]
        ext = jnp.concatenate([radial, etc, ones_col], axis=1)
        contrib = jnp.dot(ext, wext, preferred_element_type=jnp.float32)
        pre = preab_ref[pl.ds(g * 128, 128), :] + contrib
        x = _silu(pre)
        xm = jnp.dot(x, w2, preferred_element_type=jnp.float32) + b2
        m_ref[pl.ds(g * 128, 128), :] = _silu(xm)

        dc = dstt[:, g:g + 1]
        qf = jnp.floor(dc * (1.0 / 128.0))
        rf = dc - qf * 128.0
        ohq = jnp.where(qf == liota, 1.0, 0.0)
        ohr = jnp.where(rf == liota, 1.0, 0.0)
        acc = acc + lax.dot_general(ohq, ohr, (((0,), (0,)), ((), ())),
                                    preferred_element_type=jnp.float32)

    @pl.when(i == 0)
    def _():
        cnt_ref[...] = jnp.zeros_like(cnt_ref)

    cnt_ref[...] += acc


def _stage_c(preab, d2g, etg, dstg, wext, censg1, gam1, w2, b2):
    ne = preab.shape[0]
    grid = ne // C_BLK
    return pl.pallas_call(
        _stage_c_body,
        grid=(grid,),
        in_specs=[
            pl.BlockSpec((C_BLK, H), lambda i: (i, 0)),
            pl.BlockSpec((1, C_G, 128), lambda i: (i, 0, 0)),
            pl.BlockSpec((1, C_G, 128), lambda i: (i, 0, 0)),
            pl.BlockSpec((1, C_G, 128), lambda i: (i, 0, 0)),
            pl.BlockSpec((NUM_RBF + 2, H), lambda i: (0, 0)),
            pl.BlockSpec((1, NUM_RBF), lambda i: (0, 0)),
            pl.BlockSpec((1, 1), lambda i: (0, 0)),
            pl.BlockSpec((H, H), lambda i: (0, 0)),
            pl.BlockSpec((1, H), lambda i: (0, 0)),
        ],
        out_specs=[
            pl.BlockSpec((C_BLK, H), lambda i: (i, 0)),
            pl.BlockSpec((128, 128), lambda i: (0, 0)),
        ],
        out_shape=[
            jax.ShapeDtypeStruct((ne, H), jnp.float32),
            jax.ShapeDtypeStruct((128, 128), jnp.float32),
        ],
    )(preab, d2g, etg, dstg, wext, censg1, gam1, w2, b2)


def _make_stage_d(ept, e0):
    nfull = ept // CH
    nf = nfull - (nfull % 2)
    extra = nfull - nf
    tail = ept - nfull * CH

    def body(m_hbm, dst_hbm, zeros2_hbm, sums2_hbm,
             ssum, idx0, idx1, idx_t, rows0, rows1,
             sml0, sml1, ssc0, ssc1):
        cid = lax.axis_index("c")
        sid = lax.axis_index("s")
        rpt = N_PAD // NS
        rbase = sid * rpt
        idx = (idx0, idx1)
        rows = (rows0, rows1)
        sml = (sml0, sml1)
        ssc = (ssc0, ssc1)

        pltpu.sync_copy(zeros2_hbm.at[pl.ds(rbase, rpt)],
                        ssum.at[pl.ds(rbase, rpt)])
        plsc.subcore_barrier()

        lbase0 = (cid * NS + sid) * ept
        gbase0 = e0 + lbase0

        pltpu.sync_copy(dst_hbm.at[pl.ds(gbase0, CH)], idx[0])
        pltpu.async_copy(m_hbm.at[pl.ds(lbase0, CH)], rows[0], sml[0])

        def pair(g, carry):
            for b in (0, 1):
                i = g * 2 + b
                nb = 1 - b
                lbase = lbase0 + i * CH
                pltpu.make_async_copy(m_hbm.at[pl.ds(lbase, CH)], rows[b],
                                      sml[b]).wait()
                pltpu.async_copy(rows[b], ssum.at[idx[b]], ssc[b], add=True)

                def advance():
                    pltpu.sync_copy(
                        dst_hbm.at[pl.ds(gbase0 + (i + 1) * CH, CH)], idx[nb])
                    pltpu.async_copy(m_hbm.at[pl.ds(lbase0 + (i + 1) * CH, CH)],
                                     rows[nb], sml[nb])

                if b == 0:
                    @pl.when(g > 0)
                    def _():
                        pltpu.make_async_copy(rows[nb], ssum.at[idx[nb]],
                                              ssc[nb]).wait()
                    advance()
                else:
                    @pl.when(g < nf // 2 - 1)
                    def _():
                        pltpu.make_async_copy(rows[nb], ssum.at[idx[nb]],
                                              ssc[nb]).wait()
                        advance()
            return carry

        lax.fori_loop(0, nf // 2, pair, 0)

        pltpu.make_async_copy(rows[0], ssum.at[idx[0]], ssc[0]).wait()
        pltpu.make_async_copy(rows[1], ssum.at[idx[1]], ssc[1]).wait()

        if extra:
            gb = gbase0 + nf * CH
            lb = lbase0 + nf * CH
            pltpu.sync_copy(dst_hbm.at[pl.ds(gb, CH)], idx[0])
            pltpu.sync_copy(m_hbm.at[pl.ds(lb, CH)], rows[0])
            pltpu.sync_copy(rows[0], ssum.at[idx[0]], add=True)

        if tail:
            gt = gbase0 + nfull * CH
            lt = lbase0 + nfull * CH
            pltpu.sync_copy(dst_hbm.at[pl.ds(gt, tail)], idx_t)
            pltpu.sync_copy(m_hbm.at[pl.ds(lt, tail)], rows0.at[pl.ds(0, tail)])
            pltpu.sync_copy(rows0.at[pl.ds(0, tail)], ssum.at[idx_t], add=True)

        plsc.subcore_barrier()
        pltpu.sync_copy(ssum.at[pl.ds(rbase, rpt)],
                        sums2_hbm.at[pl.ds(cid * N_PAD + rbase, rpt)])

    mesh = plsc.VectorSubcoreMesh(core_axis_name="c", subcore_axis_name="s",
                                  num_cores=NC, num_subcores=NS)
    tshape = max(tail, 16)
    return pl.kernel(
        body,
        out_type=[
            jax.ShapeDtypeStruct((NC * N_PAD, H), jnp.float32),
        ],
        mesh=mesh,
        scratch_types=[
            pltpu.VMEM_SHARED((N_PAD, H), jnp.float32),
            pltpu.VMEM((CH,), jnp.int32),
            pltpu.VMEM((CH,), jnp.int32),
            pltpu.VMEM((tshape,), jnp.int32),
            pltpu.VMEM((CH, H), jnp.float32),
            pltpu.VMEM((CH, H), jnp.float32),
            pltpu.SemaphoreType.DMA,
            pltpu.SemaphoreType.DMA,
            pltpu.SemaphoreType.DMA,
            pltpu.SemaphoreType.DMA,
        ],
    )


def _stage_e_body(sa_ref, cnta_ref, h_ref, mask_ref,
                  u1a_ref, u1b_ref, u1v_ref, u2m_ref, u2v_ref,
                  g_ref, b_ref, out_ref):
    s = sa_ref[0] + sa_ref[1]
    c = cnta_ref[...]
    m_i = s / jnp.maximum(c, 1.0)
    hv = h_ref[...]
    u = _silu(jnp.dot(hv, u1a_ref[...], preferred_element_type=jnp.float32)
              + jnp.dot(m_i, u1b_ref[...], preferred_element_type=jnp.float32)
              + u1v_ref[...])
    upd = jnp.dot(u, u2m_ref[...], preferred_element_type=jnp.float32) + u2v_ref[...]
    y = hv + upd
    mu = jnp.mean(y, axis=1, keepdims=True)
    var = jnp.mean((y - mu) ** 2, axis=1, keepdims=True)
    yn = (y - mu) / jnp.sqrt(var + 1e-5) * g_ref[...] + b_ref[...]
    out_ref[...] = jnp.where(mask_ref[...] > 0.5, yn, hv)


def _stage_e(sa, cnta, h, mask, u1a, u1b, u1v, u2m, u2v, g, b):
    blk = 1000
    grid = N // blk
    return pl.pallas_call(
        _stage_e_body,
        grid=(grid,),
        in_specs=[
            pl.BlockSpec((NC, blk, H), lambda i: (0, i, 0)),
            pl.BlockSpec((blk, 1), lambda i: (i, 0)),
            pl.BlockSpec((blk, H), lambda i: (i, 0)),
            pl.BlockSpec((blk, 1), lambda i: (i, 0)),
            pl.BlockSpec((H, H), lambda i: (0, 0)),
            pl.BlockSpec((H, H), lambda i: (0, 0)),
            pl.BlockSpec((1, H), lambda i: (0, 0)),
            pl.BlockSpec((H, H), lambda i: (0, 0)),
            pl.BlockSpec((1, H), lambda i: (0, 0)),
            pl.BlockSpec((1, H), lambda i: (0, 0)),
            pl.BlockSpec((1, H), lambda i: (0, 0)),
        ],
        out_specs=pl.BlockSpec((blk, H), lambda i: (i, 0)),
        out_shape=jax.ShapeDtypeStruct((N, H), jnp.float32),
    )(sa, cnta, h, mask, u1a, u1b, u1v, u2m, u2v, g, b)


@jax.jit
def kernel(h, pos, edge_index, edge_type, node_type, centers, emb,
           W1, b1, W2, b2, U1, u1, U2, u2, ln_g, ln_b):
    src = edge_index[0].astype(jnp.int32)
    dst = edge_index[1].astype(jnp.int32)

    w1a, w1b, w1c, w1d = W1[:H], W1[H:2 * H], W1[2 * H:3 * H], W1[3 * H:]
    step = centers[1] - centers[0]
    gamma = 1.0 / jnp.maximum(step * step, 1e-6)
    sg = jnp.sqrt(gamma)
    censg1 = (centers * sg).reshape(1, NUM_RBF)
    gam1 = gamma.reshape(1, 1)

    posflat = jnp.zeros((N, POS_W), jnp.float32).at[:, :3].set(pos).reshape(-1)
    etg = edge_type.astype(jnp.float32).reshape(E // C_BLK, C_G, 128)
    dstg = dst.astype(jnp.float32).reshape(E // C_BLK, C_G, 128)

    ha, hb, embc = _stage_a(h, w1a, w1b, emb, w1c, b1.reshape(1, H))

    bf = _make_stage_b(E // NW, 0)
    preab, d2 = bf(ha, hb, posflat, src, dst)

    wext = jnp.concatenate(
        [w1d, (embc[1] - embc[0])[None, :], embc[0][None, :]], axis=0)
    m, cnt128 = _stage_c(preab, d2.reshape(E // C_BLK, C_G, 128), etg, dstg,
                         wext, censg1, gam1, W2, b2.reshape(1, H))

    zeros2 = jnp.zeros((N_PAD, H), jnp.float32)
    df = _make_stage_d(E // NW, 0)
    sa = df(m, dst, zeros2)
    if isinstance(sa, (tuple, list)):
        sa = sa[0]
    sa = sa.reshape(NC, N_PAD, H)
    cnta = cnt128.reshape(-1)[:N_PAD].reshape(N_PAD, 1)

    mask = (node_type == 1).astype(jnp.float32).reshape(N, 1)

    u1a, u1b = U1[:H], U1[H:]
    out = _stage_e(sa, cnta, h, mask, u1a, u1b,
                   u1.reshape(1, H), U2, u2.reshape(1, H),
                   ln_g.reshape(1, H), ln_b.reshape(1, H))
    return out


# speedup vs baseline: 1.0136x; 1.0136x over previous
import jax
import jax.numpy as jnp
from jax import lax
from jax.experimental import pallas as pl
from jax.experimental.pallas import tpu as pltpu
from jax.experimental.pallas import tpu_sc as plsc

N = 10000
E = 320000
H = 128
NUM_RBF = 32
N_PAD = 10240

NC, NS = 2, 16
NW = NC * NS
CH = 128
POS_W = 4

EPT1 = 4992
EPT2 = 5008
E1 = EPT1 * NW
E2 = EPT2 * NW
BLK1 = 2048
BLK2 = 2504


def _silu(x):
    return x * (1.0 / (1.0 + jnp.exp(-x)))


def _stage_a_body(h_ref, w1a_ref, w1b_ref, emb_ref, w1c_ref, b1_ref,
                  ha_ref, hb_ref, embc_ref):
    hv = h_ref[...]
    ha_ref[...] = jnp.dot(hv, w1a_ref[...], preferred_element_type=jnp.float32)
    hb_ref[...] = jnp.dot(hv, w1b_ref[...], preferred_element_type=jnp.float32)
    embc_ref[...] = (
        jnp.dot(emb_ref[...], w1c_ref[...], preferred_element_type=jnp.float32)
        + b1_ref[...])


def _stage_a(h, w1a, w1b, emb, w1c, b1):
    blk = 2000
    grid = N // blk
    return pl.pallas_call(
        _stage_a_body,
        grid=(grid,),
        in_specs=[
            pl.BlockSpec((blk, H), lambda i: (i, 0)),
            pl.BlockSpec((H, H), lambda i: (0, 0)),
            pl.BlockSpec((H, H), lambda i: (0, 0)),
            pl.BlockSpec((2, H), lambda i: (0, 0)),
            pl.BlockSpec((H, H), lambda i: (0, 0)),
            pl.BlockSpec((1, H), lambda i: (0, 0)),
        ],
        out_specs=[
            pl.BlockSpec((blk, H), lambda i: (i, 0)),
            pl.BlockSpec((blk, H), lambda i: (i, 0)),
            pl.BlockSpec((2, H), lambda i: (0, 0)),
        ],
        out_shape=[
            jax.ShapeDtypeStruct((N, H), jnp.float32),
            jax.ShapeDtypeStruct((N, H), jnp.float32),
            jax.ShapeDtypeStruct((2, H), jnp.float32),
        ],
    )(h, w1a, w1b, emb, w1c, b1)


def _make_stage_b(ept, e0):
    nfull = ept // CH
    nf = nfull - (nfull % 2)
    extra = nfull - nf
    tail = ept - nfull * CH
    ne = ept * NW

    def body(ha_hbm, hb_hbm, posflat_hbm, src_hbm, dst_hbm,
             preab_hbm, d2_hbm,
             idx_s0, idx_s1, idx_s2, idx_d0, idx_d1, idx_d2,
             idx_s_t, idx_d_t,
             rows0, rows1, rows2, d2b0, d2b1, d2b2, posv,
             sga0, sga1, sga2, sgb0, sgb1, sgb2, sst0, sst1, sst2):
        wid = lax.axis_index("c") * NS + lax.axis_index("s")
        lbase0 = wid * ept
        gbase0 = e0 + lbase0
        idx_s = (idx_s0, idx_s1, idx_s2)
        idx_d = (idx_d0, idx_d1, idx_d2)
        rows = (rows0, rows1, rows2)
        d2b = (d2b0, d2b1, d2b2)
        sga = (sga0, sga1, sga2)
        sgb = (sgb0, sgb1, sgb2)
        sst = (sst0, sst1, sst2)

        pltpu.sync_copy(posflat_hbm, posv)

        def dist2_chunk(si, di, out, n_groups):
            for g in range(n_groups):
                s16 = si[pl.ds(g * 16, 16)]
                d16 = di[pl.ds(g * 16, 16)]
                sb = s16 * POS_W
                db = d16 * POS_W
                acc = jnp.zeros((16,), jnp.float32)
                for k in range(3):
                    a = plsc.load_gather(posv, [sb + k])
                    bb = plsc.load_gather(posv, [db + k])
                    r = a - bb
                    acc = acc + r * r
                out[pl.ds(g * 16, 16)] = acc

        def issue_store(j, slot):
            base = lbase0 + j * CH
            pltpu.async_copy(rows[slot], preab_hbm.at[pl.ds(base, CH)],
                             sst[slot])
            pltpu.async_copy(d2b[slot], d2_hbm.at[pl.ds(base, CH)], sst[slot])

        def wait_store(j, slot):
            base = lbase0 + j * CH
            pltpu.make_async_copy(rows[slot], preab_hbm.at[pl.ds(base, CH)],
                                  sst[slot]).wait()
            pltpu.make_async_copy(d2b[slot], d2_hbm.at[pl.ds(base, CH)],
                                  sst[slot]).wait()

        def wait_ga(slot):
            pltpu.make_async_copy(ha_hbm.at[idx_s[slot]], rows[slot],
                                  sga[slot]).wait()

        def wait_gb(slot):
            pltpu.make_async_copy(hb_hbm.at[idx_d[slot]], rows[slot],
                                  sgb[slot]).wait()

        def load_idx_and_ga(j, slot):
            gb = gbase0 + j * CH
            pltpu.sync_copy(src_hbm.at[pl.ds(gb, CH)], idx_s[slot])
            pltpu.sync_copy(dst_hbm.at[pl.ds(gb, CH)], idx_d[slot])
            pltpu.async_copy(ha_hbm.at[idx_s[slot]], rows[slot], sga[slot])

        pltpu.sync_copy(src_hbm.at[pl.ds(gbase0, CH)], idx_s[0])
        pltpu.sync_copy(dst_hbm.at[pl.ds(gbase0, CH)], idx_d[0])
        pltpu.async_copy(ha_hbm.at[idx_s[0]], rows[0], sga[0])

        def pair(g, carry):
            for b in (0, 1):
                i = g * 2 + b
                nb = 1 - b
                if b == 0:
                    pltpu.sync_copy(
                        src_hbm.at[pl.ds(gbase0 + (i + 1) * CH, CH)],
                        idx_s[nb])
                else:
                    @pl.when(g < nf // 2 - 1)
                    def _():
                        pltpu.sync_copy(
                            src_hbm.at[pl.ds(gbase0 + (i + 1) * CH, CH)],
                            idx_s[nb])
                wait_ga(b)
                pltpu.async_copy(hb_hbm.at[idx_d[b]], rows[b], sgb[b],
                                 add=True)
                dist2_chunk(idx_s[b], idx_d[b], d2b[b], CH // 16)
                if b == 0:
                    @pl.when(g > 0)
                    def _():
                        wait_gb(nb)
                        issue_store(i - 1, nb)
                        pltpu.sync_copy(
                            dst_hbm.at[pl.ds(gbase0 + (i + 1) * CH, CH)],
                            idx_d[nb])
                        wait_store(i - 1, nb)

                    @pl.when(g == 0)
                    def _():
                        pltpu.sync_copy(
                            dst_hbm.at[pl.ds(gbase0 + (i + 1) * CH, CH)],
                            idx_d[nb])
                    pltpu.async_copy(ha_hbm.at[idx_s[nb]], rows[nb], sga[nb])
                else:
                    @pl.when(g < nf // 2 - 1)
                    def _():
                        wait_gb(nb)
                        issue_store(i - 1, nb)
                        pltpu.sync_copy(
                            dst_hbm.at[pl.ds(gbase0 + (i + 1) * CH, CH)],
                            idx_d[nb])
                        wait_store(i - 1, nb)
                        pltpu.async_copy(ha_hbm.at[idx_s[nb]], rows[nb],
                                         sga[nb])
            return carry

        lax.fori_loop(0, nf // 2, pair, 0)

        wait_gb(0)
        issue_store(nf - 2, 0)
        wait_store(nf - 2, 0)
        wait_gb(1)
        issue_store(nf - 1, 1)
        wait_store(nf - 1, 1)

        for x in range(extra):
            gb = gbase0 + (nf + x) * CH
            lb = lbase0 + (nf + x) * CH
            pltpu.sync_copy(src_hbm.at[pl.ds(gb, CH)], idx_s[0])
            pltpu.sync_copy(dst_hbm.at[pl.ds(gb, CH)], idx_d[0])
            pltpu.async_copy(ha_hbm.at[idx_s[0]], rows[0], sga[0]).wait()
            cp = pltpu.async_copy(hb_hbm.at[idx_d[0]], rows[0], sga[0],
                                  add=True)
            dist2_chunk(idx_s[0], idx_d[0], d2b[0], CH // 16)
            cp.wait()
            pltpu.sync_copy(rows[0], preab_hbm.at[pl.ds(lb, CH)])
            pltpu.sync_copy(d2b[0], d2_hbm.at[pl.ds(lb, CH)])

        if tail:
            gt = gbase0 + nfull * CH
            lt = lbase0 + nfull * CH
            pltpu.sync_copy(src_hbm.at[pl.ds(gt, tail)], idx_s_t)
            pltpu.sync_copy(dst_hbm.at[pl.ds(gt, tail)], idx_d_t)
            pltpu.async_copy(ha_hbm.at[idx_s_t], rows0.at[pl.ds(0, tail)],
                             sga0).wait()
            cp = pltpu.async_copy(hb_hbm.at[idx_d_t], rows0.at[pl.ds(0, tail)],
                                  sga0, add=True)
            dist2_chunk(idx_s_t, idx_d_t, d2b0, tail // 16)
            cp.wait()
            pltpu.sync_copy(rows0.at[pl.ds(0, tail)],
                            preab_hbm.at[pl.ds(lt, tail)])
            pltpu.sync_copy(d2b0.at[pl.ds(0, tail)], d2_hbm.at[pl.ds(lt, tail)])

    mesh = plsc.VectorSubcoreMesh(core_axis_name="c", subcore_axis_name="s",
                                  num_cores=NC, num_subcores=NS)
    tshape = max(tail, 16)
    return pl.kernel(
        body,
        out_type=[
            jax.ShapeDtypeStruct((ne, H), jnp.float32),
            jax.ShapeDtypeStruct((ne,), jnp.float32),
        ],
        mesh=mesh,
        scratch_types=[
            pltpu.VMEM((CH,), jnp.int32),
            pltpu.VMEM((CH,), jnp.int32),
            pltpu.VMEM((CH,), jnp.int32),
            pltpu.VMEM((CH,), jnp.int32),
            pltpu.VMEM((CH,), jnp.int32),
            pltpu.VMEM((CH,), jnp.int32),
            pltpu.VMEM((tshape,), jnp.int32),
            pltpu.VMEM((tshape,), jnp.int32),
            pltpu.VMEM((CH, H), jnp.float32),
            pltpu.VMEM((CH, H), jnp.float32),
            pltpu.VMEM((CH, H), jnp.float32),
            pltpu.VMEM((CH,), jnp.float32),
            pltpu.VMEM((CH,), jnp.float32),
            pltpu.VMEM((CH,), jnp.float32),
            pltpu.VMEM((N * POS_W,), jnp.float32),
            pltpu.SemaphoreType.DMA,
            pltpu.SemaphoreType.DMA,
            pltpu.SemaphoreType.DMA,
            pltpu.SemaphoreType.DMA,
            pltpu.SemaphoreType.DMA,
            pltpu.SemaphoreType.DMA,
            pltpu.SemaphoreType.DMA,
            pltpu.SemaphoreType.DMA,
            pltpu.SemaphoreType.DMA,
        ],
        compiler_params=pltpu.CompilerParams(needs_layout_passes=False),
    )


C_BLK = 6400
C_G = C_BLK // 128


def _stage_c_body(preab_ref, d2g_ref, etg_ref, dstg_ref, wext_ref,
                  censg_ref, gam_ref, w2_ref, b2_ref, m_ref, cnt_ref):
    i = pl.program_id(0)
    w2 = w2_ref[...]
    b2 = b2_ref[...]
    wext = wext_ref[...]
    censg = censg_ref[...]
    gam = gam_ref[...]
    liota = lax.broadcasted_iota(jnp.int32, (128, 128), 1).astype(jnp.float32)
    ones_col = jnp.ones((128, 1), jnp.float32)

    d2t = d2g_ref[0].T
    ett = etg_ref[0].T
    dstt = dstg_ref[0].T

    acc = jnp.zeros((128, 128), jnp.float32)
    for g in range(C_G):
        d2c = d2t[:, g:g + 1]
        dist = jnp.sqrt(d2c * gam)
        diff = dist - censg
        radial = jnp.exp(-diff * diff)
        etc = ett[:, g:g + 1]
        ext = jnp.concatenate([radial, etc, ones_col], axis=1)
        contrib = jnp.dot(ext, wext, preferred_element_type=jnp.float32)
        pre = preab_ref[pl.ds(g * 128, 128), :] + contrib
        x = _silu(pre)
        xm = jnp.dot(x, w2, preferred_element_type=jnp.float32) + b2
        m_ref[pl.ds(g * 128, 128), :] = _silu(xm)

        dc = dstt[:, g:g + 1]
        qf = jnp.floor(dc * (1.0 / 128.0))
        rf = dc - qf * 128.0
        ohq = jnp.where(qf == liota, 1.0, 0.0)
        ohr = jnp.where(rf == liota, 1.0, 0.0)
        acc = acc + lax.dot_general(ohq, ohr, (((0,), (0,)), ((), ())),
                                    preferred_element_type=jnp.float32)

    @pl.when(i == 0)
    def _():
        cnt_ref[...] = jnp.zeros_like(cnt_ref)

    cnt_ref[...] += acc


def _stage_c(preab, d2g, etg, dstg, wext, censg1, gam1, w2, b2):
    ne = preab.shape[0]
    grid = ne // C_BLK
    return pl.pallas_call(
        _stage_c_body,
        grid=(grid,),
        in_specs=[
            pl.BlockSpec((C_BLK, H), lambda i: (i, 0)),
            pl.BlockSpec((1, C_G, 128), lambda i: (i, 0, 0)),
            pl.BlockSpec((1, C_G, 128), lambda i: (i, 0, 0)),
            pl.BlockSpec((1, C_G, 128), lambda i: (i, 0, 0)),
            pl.BlockSpec((NUM_RBF + 2, H), lambda i: (0, 0)),
            pl.BlockSpec((1, NUM_RBF), lambda i: (0, 0)),
            pl.BlockSpec((1, 1), lambda i: (0, 0)),
            pl.BlockSpec((H, H), lambda i: (0, 0)),
            pl.BlockSpec((1, H), lambda i: (0, 0)),
        ],
        out_specs=[
            pl.BlockSpec((C_BLK, H), lambda i: (i, 0)),
            pl.BlockSpec((128, 128), lambda i: (0, 0)),
        ],
        out_shape=[
            jax.ShapeDtypeStruct((ne, H), jnp.float32),
            jax.ShapeDtypeStruct((128, 128), jnp.float32),
        ],
    )(preab, d2g, etg, dstg, wext, censg1, gam1, w2, b2)


def _make_stage_d(ept, e0):
    nfull = ept // CH
    nf = nfull - (nfull % 2)
    extra = nfull - nf
    tail = ept - nfull * CH

    def body(m_hbm, dst_hbm, zeros2_hbm, sums2_hbm,
             ssum, idx0, idx1, idx_t, rows0, rows1,
             sml0, sml1, ssc0, ssc1):
        cid = lax.axis_index("c")
        sid = lax.axis_index("s")
        rpt = N_PAD // NS
        rbase = sid * rpt
        idx = (idx0, idx1)
        rows = (rows0, rows1)
        sml = (sml0, sml1)
        ssc = (ssc0, ssc1)

        pltpu.sync_copy(zeros2_hbm.at[pl.ds(rbase, rpt)],
                        ssum.at[pl.ds(rbase, rpt)])
        plsc.subcore_barrier()

        lbase0 = (cid * NS + sid) * ept
        gbase0 = e0 + lbase0

        pltpu.sync_copy(dst_hbm.at[pl.ds(gbase0, CH)], idx[0])
        pltpu.async_copy(m_hbm.at[pl.ds(lbase0, CH)], rows[0], sml[0])

        def pair(g, carry):
            for b in (0, 1):
                i = g * 2 + b
                nb = 1 - b
                lbase = lbase0 + i * CH
                pltpu.make_async_copy(m_hbm.at[pl.ds(lbase, CH)], rows[b],
                                      sml[b]).wait()
                pltpu.async_copy(rows[b], ssum.at[idx[b]], ssc[b], add=True)

                def advance():
                    pltpu.sync_copy(
                        dst_hbm.at[pl.ds(gbase0 + (i + 1) * CH, CH)], idx[nb])
                    pltpu.async_copy(m_hbm.at[pl.ds(lbase0 + (i + 1) * CH, CH)],
                                     rows[nb], sml[nb])

                if b == 0:
                    @pl.when(g > 0)
                    def _():
                        pltpu.make_async_copy(rows[nb], ssum.at[idx[nb]],
                                              ssc[nb]).wait()
                    advance()
                else:
                    @pl.when(g < nf // 2 - 1)
                    def _():
                        pltpu.make_async_copy(rows[nb], ssum.at[idx[nb]],
                                              ssc[nb]).wait()
                        advance()
            return carry

        lax.fori_loop(0, nf // 2, pair, 0)

        pltpu.make_async_copy(rows[0], ssum.at[idx[0]], ssc[0]).wait()
        pltpu.make_async_copy(rows[1], ssum.at[idx[1]], ssc[1]).wait()

        if extra:
            gb = gbase0 + nf * CH
            lb = lbase0 + nf * CH
            pltpu.sync_copy(dst_hbm.at[pl.ds(gb, CH)], idx[0])
            pltpu.sync_copy(m_hbm.at[pl.ds(lb, CH)], rows[0])
            pltpu.sync_copy(rows[0], ssum.at[idx[0]], add=True)

        if tail:
            gt = gbase0 + nfull * CH
            lt = lbase0 + nfull * CH
            pltpu.sync_copy(dst_hbm.at[pl.ds(gt, tail)], idx_t)
            pltpu.sync_copy(m_hbm.at[pl.ds(lt, tail)], rows0.at[pl.ds(0, tail)])
            pltpu.sync_copy(rows0.at[pl.ds(0, tail)], ssum.at[idx_t], add=True)

        plsc.subcore_barrier()
        pltpu.sync_copy(ssum.at[pl.ds(rbase, rpt)],
                        sums2_hbm.at[pl.ds(cid * N_PAD + rbase, rpt)])

    mesh = plsc.VectorSubcoreMesh(core_axis_name="c", subcore_axis_name="s",
                                  num_cores=NC, num_subcores=NS)
    tshape = max(tail, 16)
    return pl.kernel(
        body,
        out_type=[
            jax.ShapeDtypeStruct((NC * N_PAD, H), jnp.float32),
        ],
        mesh=mesh,
        scratch_types=[
            pltpu.VMEM_SHARED((N_PAD, H), jnp.float32),
            pltpu.VMEM((CH,), jnp.int32),
            pltpu.VMEM((CH,), jnp.int32),
            pltpu.VMEM((tshape,), jnp.int32),
            pltpu.VMEM((CH, H), jnp.float32),
            pltpu.VMEM((CH, H), jnp.float32),
            pltpu.SemaphoreType.DMA,
            pltpu.SemaphoreType.DMA,
            pltpu.SemaphoreType.DMA,
            pltpu.SemaphoreType.DMA,
        ],
    )


def _stage_e_body(sa_ref, cnta_ref, h_ref, mask_ref,
                  u1a_ref, u1b_ref, u1v_ref, u2m_ref, u2v_ref,
                  g_ref, b_ref, out_ref):
    s = sa_ref[0] + sa_ref[1]
    c = cnta_ref[...]
    m_i = s / jnp.maximum(c, 1.0)
    hv = h_ref[...]
    u = _silu(jnp.dot(hv, u1a_ref[...], preferred_element_type=jnp.float32)
              + jnp.dot(m_i, u1b_ref[...], preferred_element_type=jnp.float32)
              + u1v_ref[...])
    upd = jnp.dot(u, u2m_ref[...], preferred_element_type=jnp.float32) + u2v_ref[...]
    y = hv + upd
    mu = jnp.mean(y, axis=1, keepdims=True)
    var = jnp.mean((y - mu) ** 2, axis=1, keepdims=True)
    yn = (y - mu) / jnp.sqrt(var + 1e-5) * g_ref[...] + b_ref[...]
    out_ref[...] = jnp.where(mask_ref[...] > 0.5, yn, hv)


def _stage_e(sa, cnta, h, mask, u1a, u1b, u1v, u2m, u2v, g, b):
    blk = 1000
    grid = N // blk
    return pl.pallas_call(
        _stage_e_body,
        grid=(grid,),
        in_specs=[
            pl.BlockSpec((NC, blk, H), lambda i: (0, i, 0)),
            pl.BlockSpec((blk, 1), lambda i: (i, 0)),
            pl.BlockSpec((blk, H), lambda i: (i, 0)),
            pl.BlockSpec((blk, 1), lambda i: (i, 0)),
            pl.BlockSpec((H, H), lambda i: (0, 0)),
            pl.BlockSpec((H, H), lambda i: (0, 0)),
            pl.BlockSpec((1, H), lambda i: (0, 0)),
            pl.BlockSpec((H, H), lambda i: (0, 0)),
            pl.BlockSpec((1, H), lambda i: (0, 0)),
            pl.BlockSpec((1, H), lambda i: (0, 0)),
            pl.BlockSpec((1, H), lambda i: (0, 0)),
        ],
        out_specs=pl.BlockSpec((blk, H), lambda i: (i, 0)),
        out_shape=jax.ShapeDtypeStruct((N, H), jnp.float32),
    )(sa, cnta, h, mask, u1a, u1b, u1v, u2m, u2v, g, b)


@jax.jit
def kernel(h, pos, edge_index, edge_type, node_type, centers, emb,
           W1, b1, W2, b2, U1, u1, U2, u2, ln_g, ln_b):
    src = edge_index[0].astype(jnp.int32)
    dst = edge_index[1].astype(jnp.int32)

    w1a, w1b, w1c, w1d = W1[:H], W1[H:2 * H], W1[2 * H:3 * H], W1[3 * H:]
    step = centers[1] - centers[0]
    gamma = 1.0 / jnp.maximum(step * step, 1e-6)
    sg = jnp.sqrt(gamma)
    censg1 = (centers * sg).reshape(1, NUM_RBF)
    gam1 = gamma.reshape(1, 1)

    posflat = jnp.zeros((N, POS_W), jnp.float32).at[:, :3].set(pos).reshape(-1)
    etg = edge_type.astype(jnp.float32).reshape(E // C_BLK, C_G, 128)
    dstg = dst.astype(jnp.float32).reshape(E // C_BLK, C_G, 128)

    ha, hb, embc = _stage_a(h, w1a, w1b, emb, w1c, b1.reshape(1, H))

    bf = _make_stage_b(E // NW, 0)
    preab, d2 = bf(ha, hb, posflat, src, dst)

    wext = jnp.concatenate(
        [w1d, (embc[1] - embc[0])[None, :], embc[0][None, :]], axis=0)
    m, cnt128 = _stage_c(preab, d2.reshape(E // C_BLK, C_G, 128), etg, dstg,
                         wext, censg1, gam1, W2, b2.reshape(1, H))

    zeros2 = jnp.zeros((N_PAD, H), jnp.float32)
    df = _make_stage_d(E // NW, 0)
    sa = df(m, dst, zeros2)
    if isinstance(sa, (tuple, list)):
        sa = sa[0]
    sa = sa.reshape(NC, N_PAD, H)
    cnta = cnt128.reshape(-1)[:N_PAD].reshape(N_PAD, 1)

    mask = (node_type == 1).astype(jnp.float32).reshape(N, 1)

    u1a, u1b = U1[:H], U1[H:]
    out = _stage_e(sa, cnta, h, mask, u1a, u1b,
                   u1.reshape(1, H), U2, u2.reshape(1, H),
                   ln_g.reshape(1, H), ln_b.reshape(1, H))
    return out


# speedup vs baseline: 1.0351x; 1.0212x over previous
import jax
import jax.numpy as jnp
from jax import lax
from jax.experimental import pallas as pl
from jax.experimental.pallas import tpu as pltpu
from jax.experimental.pallas import tpu_sc as plsc

N = 10000
E = 320000
H = 128
NUM_RBF = 32
N_PAD = 10240

NC, NS = 2, 16
NW = NC * NS
CH = 128
POS_W = 4

EPT1 = 4992
EPT2 = 5008
E1 = EPT1 * NW
E2 = EPT2 * NW
BLK1 = 2048
BLK2 = 2504


def _silu(x):
    return x * (1.0 / (1.0 + jnp.exp(-x)))


def _stage_a_body(h_ref, w1a_ref, w1b_ref, emb_ref, w1c_ref, b1_ref,
                  ha_ref, hb_ref, embc_ref):
    hv = h_ref[...]
    ha_ref[...] = jnp.dot(hv, w1a_ref[...], preferred_element_type=jnp.float32)
    hb_ref[...] = jnp.dot(hv, w1b_ref[...], preferred_element_type=jnp.float32)
    embc_ref[...] = (
        jnp.dot(emb_ref[...], w1c_ref[...], preferred_element_type=jnp.float32)
        + b1_ref[...])


def _stage_a(h, w1a, w1b, emb, w1c, b1):
    blk = 2000
    grid = N // blk
    return pl.pallas_call(
        _stage_a_body,
        grid=(grid,),
        in_specs=[
            pl.BlockSpec((blk, H), lambda i: (i, 0)),
            pl.BlockSpec((H, H), lambda i: (0, 0)),
            pl.BlockSpec((H, H), lambda i: (0, 0)),
            pl.BlockSpec((2, H), lambda i: (0, 0)),
            pl.BlockSpec((H, H), lambda i: (0, 0)),
            pl.BlockSpec((1, H), lambda i: (0, 0)),
        ],
        out_specs=[
            pl.BlockSpec((blk, H), lambda i: (i, 0)),
            pl.BlockSpec((blk, H), lambda i: (i, 0)),
            pl.BlockSpec((2, H), lambda i: (0, 0)),
        ],
        out_shape=[
            jax.ShapeDtypeStruct((N, H), jnp.float32),
            jax.ShapeDtypeStruct((N, H), jnp.float32),
            jax.ShapeDtypeStruct((2, H), jnp.float32),
        ],
    )(h, w1a, w1b, emb, w1c, b1)


def _make_stage_b(ept, e0):
    nfull = ept // CH
    nf = nfull - (nfull % 2)
    extra = nfull - nf
    tail = ept - nfull * CH
    ne = ept * NW

    def body(ha_hbm, hb_hbm, posflat_hbm, src_hbm, dst_hbm,
             preab_hbm, d2_hbm,
             idx_s0, idx_s1, idx_s2, idx_d0, idx_d1, idx_d2,
             idx_s_t, idx_d_t,
             rows0, rows1, rows2, d2b0, d2b1, d2b2, posv,
             sga0, sga1, sga2, sgb0, sgb1, sgb2, sst0, sst1, sst2):
        wid = lax.axis_index("c") * NS + lax.axis_index("s")
        lbase0 = wid * ept
        gbase0 = e0 + lbase0
        idx_s = (idx_s0, idx_s1, idx_s2)
        idx_d = (idx_d0, idx_d1, idx_d2)
        rows = (rows0, rows1, rows2)
        d2b = (d2b0, d2b1, d2b2)
        sga = (sga0, sga1, sga2)
        sgb = (sgb0, sgb1, sgb2)
        sst = (sst0, sst1, sst2)

        pltpu.sync_copy(posflat_hbm, posv)

        def dist2_chunk(si, di, out, n_groups):
            for g in range(n_groups):
                s16 = si[pl.ds(g * 16, 16)]
                d16 = di[pl.ds(g * 16, 16)]
                sb = s16 * POS_W
                db = d16 * POS_W
                acc = jnp.zeros((16,), jnp.float32)
                for k in range(3):
                    a = plsc.load_gather(posv, [sb + k])
                    bb = plsc.load_gather(posv, [db + k])
                    r = a - bb
                    acc = acc + r * r
                out[pl.ds(g * 16, 16)] = acc

        def issue_store(j, slot):
            base = lbase0 + j * CH
            pltpu.async_copy(rows[slot], preab_hbm.at[pl.ds(base, CH)],
                             sst[slot])
            pltpu.async_copy(d2b[slot], d2_hbm.at[pl.ds(base, CH)], sst[slot])

        def wait_store(j, slot):
            base = lbase0 + j * CH
            pltpu.make_async_copy(rows[slot], preab_hbm.at[pl.ds(base, CH)],
                                  sst[slot]).wait()
            pltpu.make_async_copy(d2b[slot], d2_hbm.at[pl.ds(base, CH)],
                                  sst[slot]).wait()

        def wait_ga(slot):
            pltpu.make_async_copy(ha_hbm.at[idx_s[slot]], rows[slot],
                                  sga[slot]).wait()

        def wait_gb(slot):
            pltpu.make_async_copy(hb_hbm.at[idx_d[slot]], rows[slot],
                                  sgb[slot]).wait()

        def load_idx_and_ga(j, slot):
            gb = gbase0 + j * CH
            pltpu.sync_copy(src_hbm.at[pl.ds(gb, CH)], idx_s[slot])
            pltpu.sync_copy(dst_hbm.at[pl.ds(gb, CH)], idx_d[slot])
            pltpu.async_copy(ha_hbm.at[idx_s[slot]], rows[slot], sga[slot])

        pltpu.sync_copy(src_hbm.at[pl.ds(gbase0, CH)], idx_s[0])
        pltpu.sync_copy(dst_hbm.at[pl.ds(gbase0, CH)], idx_d[0])
        pltpu.async_copy(ha_hbm.at[idx_s[0]], rows[0], sga[0])

        def pair(g, carry):
            for b in (0, 1):
                i = g * 2 + b
                nb = 1 - b
                if b == 0:
                    pltpu.sync_copy(
                        src_hbm.at[pl.ds(gbase0 + (i + 1) * CH, CH)],
                        idx_s[nb])
                else:
                    @pl.when(g < nf // 2 - 1)
                    def _():
                        pltpu.sync_copy(
                            src_hbm.at[pl.ds(gbase0 + (i + 1) * CH, CH)],
                            idx_s[nb])
                wait_ga(b)
                pltpu.async_copy(hb_hbm.at[idx_d[b]], rows[b], sgb[b],
                                 add=True)
                dist2_chunk(idx_s[b], idx_d[b], d2b[b], CH // 16)
                if b == 0:
                    @pl.when(g > 0)
                    def _():
                        wait_gb(nb)
                        issue_store(i - 1, nb)
                        pltpu.sync_copy(
                            dst_hbm.at[pl.ds(gbase0 + (i + 1) * CH, CH)],
                            idx_d[nb])
                        wait_store(i - 1, nb)

                    @pl.when(g == 0)
                    def _():
                        pltpu.sync_copy(
                            dst_hbm.at[pl.ds(gbase0 + (i + 1) * CH, CH)],
                            idx_d[nb])
                    pltpu.async_copy(ha_hbm.at[idx_s[nb]], rows[nb], sga[nb])
                else:
                    @pl.when(g < nf // 2 - 1)
                    def _():
                        wait_gb(nb)
                        issue_store(i - 1, nb)
                        pltpu.sync_copy(
                            dst_hbm.at[pl.ds(gbase0 + (i + 1) * CH, CH)],
                            idx_d[nb])
                        wait_store(i - 1, nb)
                        pltpu.async_copy(ha_hbm.at[idx_s[nb]], rows[nb],
                                         sga[nb])
            return carry

        lax.fori_loop(0, nf // 2, pair, 0)

        wait_gb(0)
        issue_store(nf - 2, 0)
        wait_store(nf - 2, 0)
        wait_gb(1)
        issue_store(nf - 1, 1)
        wait_store(nf - 1, 1)

        for x in range(extra):
            gb = gbase0 + (nf + x) * CH
            lb = lbase0 + (nf + x) * CH
            pltpu.sync_copy(src_hbm.at[pl.ds(gb, CH)], idx_s[0])
            pltpu.sync_copy(dst_hbm.at[pl.ds(gb, CH)], idx_d[0])
            pltpu.async_copy(ha_hbm.at[idx_s[0]], rows[0], sga[0]).wait()
            cp = pltpu.async_copy(hb_hbm.at[idx_d[0]], rows[0], sga[0],
                                  add=True)
            dist2_chunk(idx_s[0], idx_d[0], d2b[0], CH // 16)
            cp.wait()
            pltpu.sync_copy(rows[0], preab_hbm.at[pl.ds(lb, CH)])
            pltpu.sync_copy(d2b[0], d2_hbm.at[pl.ds(lb, CH)])

        if tail:
            gt = gbase0 + nfull * CH
            lt = lbase0 + nfull * CH
            pltpu.sync_copy(src_hbm.at[pl.ds(gt, tail)], idx_s_t)
            pltpu.sync_copy(dst_hbm.at[pl.ds(gt, tail)], idx_d_t)
            pltpu.async_copy(ha_hbm.at[idx_s_t], rows0.at[pl.ds(0, tail)],
                             sga0).wait()
            cp = pltpu.async_copy(hb_hbm.at[idx_d_t], rows0.at[pl.ds(0, tail)],
                                  sga0, add=True)
            dist2_chunk(idx_s_t, idx_d_t, d2b0, tail // 16)
            cp.wait()
            pltpu.sync_copy(rows0.at[pl.ds(0, tail)],
                            preab_hbm.at[pl.ds(lt, tail)])
            pltpu.sync_copy(d2b0.at[pl.ds(0, tail)], d2_hbm.at[pl.ds(lt, tail)])

    mesh = plsc.VectorSubcoreMesh(core_axis_name="c", subcore_axis_name="s",
                                  num_cores=NC, num_subcores=NS)
    tshape = max(tail, 16)
    return pl.kernel(
        body,
        out_type=[
            jax.ShapeDtypeStruct((ne, H), jnp.float32),
            jax.ShapeDtypeStruct((ne,), jnp.float32),
        ],
        mesh=mesh,
        scratch_types=[
            pltpu.VMEM((CH,), jnp.int32),
            pltpu.VMEM((CH,), jnp.int32),
            pltpu.VMEM((CH,), jnp.int32),
            pltpu.VMEM((CH,), jnp.int32),
            pltpu.VMEM((CH,), jnp.int32),
            pltpu.VMEM((CH,), jnp.int32),
            pltpu.VMEM((tshape,), jnp.int32),
            pltpu.VMEM((tshape,), jnp.int32),
            pltpu.VMEM((CH, H), jnp.float32),
            pltpu.VMEM((CH, H), jnp.float32),
            pltpu.VMEM((CH, H), jnp.float32),
            pltpu.VMEM((CH,), jnp.float32),
            pltpu.VMEM((CH,), jnp.float32),
            pltpu.VMEM((CH,), jnp.float32),
            pltpu.VMEM((N * POS_W,), jnp.float32),
            pltpu.SemaphoreType.DMA,
            pltpu.SemaphoreType.DMA,
            pltpu.SemaphoreType.DMA,
            pltpu.SemaphoreType.DMA,
            pltpu.SemaphoreType.DMA,
            pltpu.SemaphoreType.DMA,
            pltpu.SemaphoreType.DMA,
            pltpu.SemaphoreType.DMA,
            pltpu.SemaphoreType.DMA,
        ],
        compiler_params=pltpu.CompilerParams(needs_layout_passes=False),
    )


C_BLK = 12800
C_G = C_BLK // 128


def _stage_c_body(preab_ref, d2g_ref, etg_ref, dstg_ref, wext_ref,
                  censg_ref, gam_ref, w2_ref, b2_ref, m_ref, cnt_ref):
    i = pl.program_id(0)
    w2 = w2_ref[...]
    b2 = b2_ref[...]
    wext = wext_ref[...]
    censg = censg_ref[...]
    gam = gam_ref[...]
    liota = lax.broadcasted_iota(jnp.int32, (128, 128), 1).astype(jnp.float32)
    ones_col = jnp.ones((128, 1), jnp.float32)

    d2t = d2g_ref[0].T
    ett = etg_ref[0].T
    dstt = dstg_ref[0].T

    acc = jnp.zeros((128, 128), jnp.float32)
    for g in range(C_G):
        d2c = d2t[:, g:g + 1]
        dist = jnp.sqrt(d2c * gam)
        diff = dist - censg
        radial = jnp.exp(-diff * diff)
        etc = ett[:, g:g + 1]
        ext = jnp.concatenate([radial, etc, ones_col], axis=1)
        contrib = jnp.dot(ext, wext, preferred_element_type=jnp.float32)
        pre = preab_ref[pl.ds(g * 128, 128), :] + contrib
        x = _silu(pre)
        xm = jnp.dot(x, w2, preferred_element_type=jnp.float32) + b2
        m_ref[pl.ds(g * 128, 128), :] = _silu(xm)

        dc = dstt[:, g:g + 1]
        qf = jnp.floor(dc * (1.0 / 128.0))
        rf = dc - qf * 128.0
        ohq = jnp.where(qf == liota, 1.0, 0.0)
        ohr = jnp.where(rf == liota, 1.0, 0.0)
        acc = acc + lax.dot_general(ohq, ohr, (((0,), (0,)), ((), ())),
                                    preferred_element_type=jnp.float32)

    @pl.when(i == 0)
    def _():
        cnt_ref[...] = jnp.zeros_like(cnt_ref)

    cnt_ref[...] += acc


def _stage_c(preab, d2g, etg, dstg, wext, censg1, gam1, w2, b2):
    ne = preab.shape[0]
    grid = ne // C_BLK
    return pl.pallas_call(
        _stage_c_body,
        grid=(grid,),
        in_specs=[
            pl.BlockSpec((C_BLK, H), lambda i: (i, 0)),
            pl.BlockSpec((1, C_G, 128), lambda i: (i, 0, 0)),
            pl.BlockSpec((1, C_G, 128), lambda i: (i, 0, 0)),
            pl.BlockSpec((1, C_G, 128), lambda i: (i, 0, 0)),
            pl.BlockSpec((NUM_RBF + 2, H), lambda i: (0, 0)),
            pl.BlockSpec((1, NUM_RBF), lambda i: (0, 0)),
            pl.BlockSpec((1, 1), lambda i: (0, 0)),
            pl.BlockSpec((H, H), lambda i: (0, 0)),
            pl.BlockSpec((1, H), lambda i: (0, 0)),
        ],
        out_specs=[
            pl.BlockSpec((C_BLK, H), lambda i: (i, 0)),
            pl.BlockSpec((128, 128), lambda i: (0, 0)),
        ],
        out_shape=[
            jax.ShapeDtypeStruct((ne, H), jnp.float32),
            jax.ShapeDtypeStruct((128, 128), jnp.float32),
        ],
    )(preab, d2g, etg, dstg, wext, censg1, gam1, w2, b2)


def _make_stage_d(ept, e0):
    nfull = ept // CH
    nf = nfull - (nfull % 2)
    extra = nfull - nf
    tail = ept - nfull * CH

    def body(m_hbm, dst_hbm, zeros2_hbm, sums2_hbm,
             ssum, idx0, idx1, idx_t, rows0, rows1,
             sml0, sml1, ssc0, ssc1):
        cid = lax.axis_index("c")
        sid = lax.axis_index("s")
        rpt = N_PAD // NS
        rbase = sid * rpt
        idx = (idx0, idx1)
        rows = (rows0, rows1)
        sml = (sml0, sml1)
        ssc = (ssc0, ssc1)

        pltpu.sync_copy(zeros2_hbm.at[pl.ds(rbase, rpt)],
                        ssum.at[pl.ds(rbase, rpt)])
        plsc.subcore_barrier()

        lbase0 = (cid * NS + sid) * ept
        gbase0 = e0 + lbase0

        pltpu.sync_copy(dst_hbm.at[pl.ds(gbase0, CH)], idx[0])
        pltpu.async_copy(m_hbm.at[pl.ds(lbase0, CH)], rows[0], sml[0])

        def pair(g, carry):
            for b in (0, 1):
                i = g * 2 + b
                nb = 1 - b
                lbase = lbase0 + i * CH
                pltpu.make_async_copy(m_hbm.at[pl.ds(lbase, CH)], rows[b],
                                      sml[b]).wait()
                pltpu.async_copy(rows[b], ssum.at[idx[b]], ssc[b], add=True)

                def advance():
                    pltpu.sync_copy(
                        dst_hbm.at[pl.ds(gbase0 + (i + 1) * CH, CH)], idx[nb])
                    pltpu.async_copy(m_hbm.at[pl.ds(lbase0 + (i + 1) * CH, CH)],
                                     rows[nb], sml[nb])

                if b == 0:
                    @pl.when(g > 0)
                    def _():
                        pltpu.make_async_copy(rows[nb], ssum.at[idx[nb]],
                                              ssc[nb]).wait()
                    advance()
                else:
                    @pl.when(g < nf // 2 - 1)
                    def _():
                        pltpu.make_async_copy(rows[nb], ssum.at[idx[nb]],
                                              ssc[nb]).wait()
                        advance()
            return carry

        lax.fori_loop(0, nf // 2, pair, 0)

        pltpu.make_async_copy(rows[0], ssum.at[idx[0]], ssc[0]).wait()
        pltpu.make_async_copy(rows[1], ssum.at[idx[1]], ssc[1]).wait()

        if extra:
            gb = gbase0 + nf * CH
            lb = lbase0 + nf * CH
            pltpu.sync_copy(dst_hbm.at[pl.ds(gb, CH)], idx[0])
            pltpu.sync_copy(m_hbm.at[pl.ds(lb, CH)], rows[0])
            pltpu.sync_copy(rows[0], ssum.at[idx[0]], add=True)

        if tail:
            gt = gbase0 + nfull * CH
            lt = lbase0 + nfull * CH
            pltpu.sync_copy(dst_hbm.at[pl.ds(gt, tail)], idx_t)
            pltpu.sync_copy(m_hbm.at[pl.ds(lt, tail)], rows0.at[pl.ds(0, tail)])
            pltpu.sync_copy(rows0.at[pl.ds(0, tail)], ssum.at[idx_t], add=True)

        plsc.subcore_barrier()
        pltpu.sync_copy(ssum.at[pl.ds(rbase, rpt)],
                        sums2_hbm.at[pl.ds(cid * N_PAD + rbase, rpt)])

    mesh = plsc.VectorSubcoreMesh(core_axis_name="c", subcore_axis_name="s",
                                  num_cores=NC, num_subcores=NS)
    tshape = max(tail, 16)
    return pl.kernel(
        body,
        out_type=[
            jax.ShapeDtypeStruct((NC * N_PAD, H), jnp.float32),
        ],
        mesh=mesh,
        scratch_types=[
            pltpu.VMEM_SHARED((N_PAD, H), jnp.float32),
            pltpu.VMEM((CH,), jnp.int32),
            pltpu.VMEM((CH,), jnp.int32),
            pltpu.VMEM((tshape,), jnp.int32),
            pltpu.VMEM((CH, H), jnp.float32),
            pltpu.VMEM((CH, H), jnp.float32),
            pltpu.SemaphoreType.DMA,
            pltpu.SemaphoreType.DMA,
            pltpu.SemaphoreType.DMA,
            pltpu.SemaphoreType.DMA,
        ],
    )


def _stage_e_body(sa_ref, cnta_ref, h_ref, mask_ref,
                  u1a_ref, u1b_ref, u1v_ref, u2m_ref, u2v_ref,
                  g_ref, b_ref, out_ref):
    s = sa_ref[0] + sa_ref[1]
    c = cnta_ref[...]
    m_i = s / jnp.maximum(c, 1.0)
    hv = h_ref[...]
    u = _silu(jnp.dot(hv, u1a_ref[...], preferred_element_type=jnp.float32)
              + jnp.dot(m_i, u1b_ref[...], preferred_element_type=jnp.float32)
              + u1v_ref[...])
    upd = jnp.dot(u, u2m_ref[...], preferred_element_type=jnp.float32) + u2v_ref[...]
    y = hv + upd
    mu = jnp.mean(y, axis=1, keepdims=True)
    var = jnp.mean((y - mu) ** 2, axis=1, keepdims=True)
    yn = (y - mu) / jnp.sqrt(var + 1e-5) * g_ref[...] + b_ref[...]
    out_ref[...] = jnp.where(mask_ref[...] > 0.5, yn, hv)


def _stage_e(sa, cnta, h, mask, u1a, u1b, u1v, u2m, u2v, g, b):
    blk = 1000
    grid = N // blk
    return pl.pallas_call(
        _stage_e_body,
        grid=(grid,),
        in_specs=[
            pl.BlockSpec((NC, blk, H), lambda i: (0, i, 0)),
            pl.BlockSpec((blk, 1), lambda i: (i, 0)),
            pl.BlockSpec((blk, H), lambda i: (i, 0)),
            pl.BlockSpec((blk, 1), lambda i: (i, 0)),
            pl.BlockSpec((H, H), lambda i: (0, 0)),
            pl.BlockSpec((H, H), lambda i: (0, 0)),
            pl.BlockSpec((1, H), lambda i: (0, 0)),
            pl.BlockSpec((H, H), lambda i: (0, 0)),
            pl.BlockSpec((1, H), lambda i: (0, 0)),
            pl.BlockSpec((1, H), lambda i: (0, 0)),
            pl.BlockSpec((1, H), lambda i: (0, 0)),
        ],
        out_specs=pl.BlockSpec((blk, H), lambda i: (i, 0)),
        out_shape=jax.ShapeDtypeStruct((N, H), jnp.float32),
    )(sa, cnta, h, mask, u1a, u1b, u1v, u2m, u2v, g, b)


@jax.jit
def kernel(h, pos, edge_index, edge_type, node_type, centers, emb,
           W1, b1, W2, b2, U1, u1, U2, u2, ln_g, ln_b):
    src = edge_index[0].astype(jnp.int32)
    dst = edge_index[1].astype(jnp.int32)

    w1a, w1b, w1c, w1d = W1[:H], W1[H:2 * H], W1[2 * H:3 * H], W1[3 * H:]
    step = centers[1] - centers[0]
    gamma = 1.0 / jnp.maximum(step * step, 1e-6)
    sg = jnp.sqrt(gamma)
    censg1 = (centers * sg).reshape(1, NUM_RBF)
    gam1 = gamma.reshape(1, 1)

    posflat = jnp.zeros((N, POS_W), jnp.float32).at[:, :3].set(pos).reshape(-1)
    etg = edge_type.astype(jnp.float32).reshape(E // C_BLK, C_G, 128)
    dstg = dst.astype(jnp.float32).reshape(E // C_BLK, C_G, 128)

    ha, hb, embc = _stage_a(h, w1a, w1b, emb, w1c, b1.reshape(1, H))

    bf = _make_stage_b(E // NW, 0)
    preab, d2 = bf(ha, hb, posflat, src, dst)

    wext = jnp.concatenate(
        [w1d, (embc[1] - embc[0])[None, :], embc[0][None, :]], axis=0)
    m, cnt128 = _stage_c(preab, d2.reshape(E // C_BLK, C_G, 128), etg, dstg,
                         wext, censg1, gam1, W2, b2.reshape(1, H))

    zeros2 = jnp.zeros((N_PAD, H), jnp.float32)
    df = _make_stage_d(E // NW, 0)
    sa = df(m, dst, zeros2)
    if isinstance(sa, (tuple, list)):
        sa = sa[0]
    sa = sa.reshape(NC, N_PAD, H)
    cnta = cnt128.reshape(-1)[:N_PAD].reshape(N_PAD, 1)

    mask = (node_type == 1).astype(jnp.float32).reshape(N, 1)

    u1a, u1b = U1[:H], U1[H:]
    out = _stage_e(sa, cnta, h, mask, u1a, u1b,
                   u1.reshape(1, H), U2, u2.reshape(1, H),
                   ln_g.reshape(1, H), ln_b.reshape(1, H))
    return out
